# baseline (speedup 1.0000x reference)
import functools

import jax
import jax.numpy as jnp
from jax import lax
from jax.experimental import pallas as pl
from jax.experimental.pallas import tpu as pltpu
from jax.experimental.pallas import tpu_sc as plsc

_B, _N, _M, _C = 16, 4096, 2048, 128

_S = 128
_U = 4
_NS = _N // _S
_G = 128


_H = 8


def _fps_body(x_ref, y_ref, z_ref, idx_ref, dist_a, dist_b):
    lane_s = lax.broadcasted_iota(jnp.int32, (_H, _S), 1)
    lane_g = lax.broadcasted_iota(jnp.int32, (_H, _G), 1)
    dist_a[...] = jnp.full((_H, _N), 1e10, jnp.float32)
    dist_b[...] = jnp.full((_H, _N), 1e10, jnp.float32)

    def group_step(dref, r, j, f, cx, cy, cz, ia):
        rows = pl.ds(r, _H)
        ia = jnp.where(lane_g == j, f, ia)
        neg = jnp.full((_H, _S), -1.0, jnp.float32)
        zi = jnp.zeros((_H, _S), jnp.int32)
        zf = jnp.zeros((_H, _S), jnp.float32)
        rv0, rg0, rx0, ry0, rz0 = neg, zi, zf, zf, zf
        rv1, rg1, rx1, ry1, rz1 = neg, zi, zf, zf, zf
        for k in range(_NS):
            sl = pl.ds(k * _S, _S)
            xk = x_ref[rows, sl]
            yk = y_ref[rows, sl]
            zk = z_ref[rows, sl]
            dxk = xk - cx
            dyk = yk - cy
            dzk = zk - cz
            dk = (dxk * dxk + dzk * dzk) + dyk * dyk
            ndk = jnp.minimum(dref[:, sl], dk)
            dref[:, sl] = ndk
            gk = lane_s + (k * _S)
            if k % 2 == 0:
                take = ndk > rv0
                rv0 = jnp.maximum(rv0, ndk)
                rg0 = jnp.where(take, gk, rg0)
                rx0 = jnp.where(take, xk, rx0)
                ry0 = jnp.where(take, yk, ry0)
                rz0 = jnp.where(take, zk, rz0)
            else:
                take = ndk > rv1
                rv1 = jnp.maximum(rv1, ndk)
                rg1 = jnp.where(take, gk, rg1)
                rx1 = jnp.where(take, xk, rx1)
                ry1 = jnp.where(take, yk, ry1)
                rz1 = jnp.where(take, zk, rz1)
        def fold(tk, a, b):
            return jnp.where(tk, b, a)

        rv, rg, rx, ry, rz = rv0, rg0, rx0, ry0, rz0
        qv, qg, qx, qy, qz = rv1, rg1, rx1, ry1, rz1
        w = _S
        while True:
            tk = (qv > rv) | ((qv == rv) & (qg < rg))
            rv = fold(tk, rv, qv)
            rg = fold(tk, rg, qg)
            rx = fold(tk, rx, qx)
            ry = fold(tk, ry, qy)
            rz = fold(tk, rz, qz)
            w //= 2
            if w == 0:
                break
            qv, rv = rv[:, w:], rv[:, :w]
            qg, rg = rg[:, w:], rg[:, :w]
            qx, rx = rx[:, w:], rx[:, :w]
            qy, ry = ry[:, w:], ry[:, :w]
            qz, rz = rz[:, w:], rz[:, :w]
        return (rg, rx, ry, rz, ia)

    def step(j, st):
        a, b = st
        return (group_step(dist_a, 0, j, *a), group_step(dist_b, _H, j, *b))

    def outer(gi, st):
        a, b = st
        zi = jnp.zeros((_H, _G), jnp.int32)
        (fa, cxa, cya, cza, iaa), (fb, cxb, cyb, czb, iab) = lax.fori_loop(
            0, _G, step, ((*a, zi), (*b, zi)))
        base = pl.ds(pl.multiple_of(gi * _G, _G), _G)
        idx_ref[pl.ds(0, _H), base] = iaa
        idx_ref[pl.ds(_H, _H), base] = iab
        return ((fa, cxa, cya, cza), (fb, cxb, cyb, czb))

    zi1 = jnp.zeros((_H, 1), jnp.int32)
    lax.fori_loop(
        0, _M // _G, outer,
        ((zi1, x_ref[pl.ds(0, _H), 0:1], y_ref[pl.ds(0, _H), 0:1],
          z_ref[pl.ds(0, _H), 0:1]),
         (zi1, x_ref[pl.ds(_H, _H), 0:1], y_ref[pl.ds(_H, _H), 0:1],
          z_ref[pl.ds(_H, _H), 0:1])))


def _fps(x, y, z):
    return pl.pallas_call(
        _fps_body,
        out_shape=jax.ShapeDtypeStruct((_B, _M), jnp.int32),
        scratch_shapes=[pltpu.VMEM((_H, _N), jnp.float32),
                        pltpu.VMEM((_H, _N), jnp.float32)],
    )(x, y, z)


_NW = 32
_CW = _C // _NW
_MH = _M // 2


def _gather_body(feats_hbm, idx_hbm, xyz_hbm, outf_hbm, outx_hbm,
                 idx_v, feat_v, out_v, xyz_v, outx_v):
    wid = lax.axis_index("s") * 2 + lax.axis_index("c")
    c0 = wid * _CW
    lane16 = lax.iota(jnp.int32, 16)
    for b in range(_B):
        pltpu.sync_copy(idx_hbm.at[b], idx_v)
        pltpu.sync_copy(feats_hbm.at[b, pl.ds(c0, _CW)], feat_v)
        for c in range(_CW):
            cvec = jnp.full((16,), c, jnp.int32)

            def inner(jj, carry):
                for u in range(4):
                    off = jj * 64 + u * 16
                    iv = idx_v[pl.ds(off, 16)]
                    out_v[c, pl.ds(off, 16)] = plsc.load_gather(
                        feat_v, [cvec, iv])
                return carry

            lax.fori_loop(0, _M // 64, inner, 0)
        pltpu.sync_copy(out_v, outf_hbm.at[b, pl.ds(c0, _CW)])
    bx = wid // 2
    h = wid % 2
    pltpu.sync_copy(idx_hbm.at[bx], idx_v)
    pltpu.sync_copy(xyz_hbm.at[bx], xyz_v)

    def xinner(jj, carry):
        for u in range(2):
            off = jj * 32 + u * 16
            iv = idx_v[pl.ds(h * _MH + off, 16)]
            iv3 = iv * 3
            mv3 = (lane16 + off) * 3
            for k in range(3):
                vals = plsc.load_gather(xyz_v, [iv3 + k])
                plsc.store_scatter(outx_v, [mv3 + k], vals)
        return carry

    lax.fori_loop(0, _MH // 32, xinner, 0)
    pltpu.sync_copy(outx_v, outx_hbm.at[bx, pl.ds(h * (_MH * 3), _MH * 3)])


def _gather(feats, idx, xyz):
    mesh = plsc.VectorSubcoreMesh(core_axis_name="c", subcore_axis_name="s")
    return pl.kernel(
        _gather_body,
        out_type=(
            jax.ShapeDtypeStruct((_B, _C, _M), jnp.float32),
            jax.ShapeDtypeStruct((_B, _M * 3), jnp.float32),
        ),
        mesh=mesh,
        compiler_params=pltpu.CompilerParams(needs_layout_passes=False),
        scratch_types=[
            pltpu.VMEM((_M,), jnp.int32),
            pltpu.VMEM((_CW, _N), jnp.float32),
            pltpu.VMEM((_CW, _M), jnp.float32),
            pltpu.VMEM((_N * 3,), jnp.float32),
            pltpu.VMEM((_MH * 3,), jnp.float32),
        ],
    )(feats, idx, jnp.reshape(xyz, (_B, _N * 3)))


def kernel(feats, xyz):
    xt = jnp.transpose(xyz, (2, 0, 1))
    idx = _fps(xt[0], xt[1], xt[2])
    new_feats, new_xyz_flat = _gather(feats, idx, xyz)
    return (new_feats, jnp.reshape(new_xyz_flat, (_B, _M, 3)))


# speedup vs baseline: 1.5227x; 1.5227x over previous
import functools

import jax
import jax.numpy as jnp
from jax import lax
from jax.experimental import pallas as pl
from jax.experimental.pallas import tpu as pltpu
from jax.experimental.pallas import tpu_sc as plsc

_B, _N, _M, _C = 16, 4096, 2048, 128


_G = 128


def _fps_body(x_ref, y_ref, z_ref, idx_ref, sx_ref, sy_ref, sz_ref, dist_ref):
    lane = lax.broadcasted_iota(jnp.int32, (_B, _N), 1)
    lane_g = lax.broadcasted_iota(jnp.int32, (_B, _G), 1)
    dist_ref[...] = jnp.full((_B, _N), 1e10, jnp.float32)

    def inner(j, st):
        f, ia, xa, ya, za = st
        x = x_ref[...]
        y = y_ref[...]
        z = z_ref[...]
        oh = lane == f
        cx = jnp.sum(jnp.where(oh, x, 0.0), axis=1, keepdims=True)
        cy = jnp.sum(jnp.where(oh, y, 0.0), axis=1, keepdims=True)
        cz = jnp.sum(jnp.where(oh, z, 0.0), axis=1, keepdims=True)
        mj = lane_g == j
        ia = jnp.where(mj, f, ia)
        xa = jnp.where(mj, cx, xa)
        ya = jnp.where(mj, cy, ya)
        za = jnp.where(mj, cz, za)
        dx = x - cx
        dy = y - cy
        dz = z - cz
        d = (dx * dx + dz * dz) + dy * dy
        nd = jnp.minimum(dist_ref[...], d)
        dist_ref[...] = nd
        mx = jnp.max(nd, axis=1, keepdims=True)
        fn = jnp.min(jnp.where(nd == mx, lane, _N), axis=1, keepdims=True)
        return (fn, ia, xa, ya, za)

    def outer(g, f):
        zi = jnp.zeros((_B, _G), jnp.int32)
        zf = jnp.zeros((_B, _G), jnp.float32)
        f, ia, xa, ya, za = lax.fori_loop(0, _G, inner, (f, zi, zf, zf, zf))
        base = pl.multiple_of(g * _G, _G)
        idx_ref[:, pl.ds(base, _G)] = ia
        sx_ref[:, pl.ds(base, _G)] = xa
        sy_ref[:, pl.ds(base, _G)] = ya
        sz_ref[:, pl.ds(base, _G)] = za
        return f

    lax.fori_loop(0, _M // _G, outer, jnp.zeros((_B, 1), jnp.int32))


def _fps(x, y, z):
    return pl.pallas_call(
        _fps_body,
        out_shape=(
            jax.ShapeDtypeStruct((_B, _M), jnp.int32),
            jax.ShapeDtypeStruct((_B, _M), jnp.float32),
            jax.ShapeDtypeStruct((_B, _M), jnp.float32),
            jax.ShapeDtypeStruct((_B, _M), jnp.float32),
        ),
        scratch_shapes=[pltpu.VMEM((_B, _N), jnp.float32)],
    )(x, y, z)


_NW = 32
_CW = _C // _NW


def _gather_body(feats_hbm, idx_hbm, out_hbm, idx_v, feat_v, out_v):
    wid = lax.axis_index("s") * 2 + lax.axis_index("c")
    c0 = wid * _CW
    for b in range(_B):
        pltpu.sync_copy(idx_hbm.at[b], idx_v)
        pltpu.sync_copy(feats_hbm.at[b, pl.ds(c0, _CW)], feat_v)
        for c in range(_CW):
            cvec = jnp.full((16,), c, jnp.int32)

            def inner(jj, carry):
                for u in range(4):
                    off = jj * 64 + u * 16
                    iv = idx_v[pl.ds(off, 16)]
                    out_v[c, pl.ds(off, 16)] = plsc.load_gather(
                        feat_v, [cvec, iv])
                return carry

            lax.fori_loop(0, _M // 64, inner, 0)
        pltpu.sync_copy(out_v, out_hbm.at[b, pl.ds(c0, _CW)])


def _gather(feats, idx):
    mesh = plsc.VectorSubcoreMesh(core_axis_name="c", subcore_axis_name="s")
    return pl.kernel(
        _gather_body,
        out_type=jax.ShapeDtypeStruct((_B, _C, _M), jnp.float32),
        mesh=mesh,
        compiler_params=pltpu.CompilerParams(needs_layout_passes=False),
        scratch_types=[
            pltpu.VMEM((_M,), jnp.int32),
            pltpu.VMEM((_CW, _N), jnp.float32),
            pltpu.VMEM((_CW, _M), jnp.float32),
        ],
    )(feats, idx)


def kernel(feats, xyz):
    xt = jnp.transpose(xyz, (2, 0, 1))
    idx, sx, sy, sz = _fps(xt[0], xt[1], xt[2])
    new_xyz = jnp.stack([sx, sy, sz], axis=-1)
    new_feats = _gather(feats, idx)
    return (new_feats, new_xyz)
